# baseline (speedup 1.0000x reference)
import functools

import jax
import jax.numpy as jnp
from jax import lax
from jax.experimental import pallas as pl
from jax.experimental.pallas import tpu as pltpu
from jax.experimental.pallas import tpu_sc as plsc

VOCAB = 1000
CPAD = 1024
B, T = 1024, 200
N_TOK = B * T

_NC, _NS = 2, 16
_NW = _NC * _NS
_TOK_PER_W = N_TOK // _NW

_CHUNK = 40
_NCHUNK = _TOK_PER_W // _CHUNK

_LCHUNK = 128
_NLCH = _TOK_PER_W // _LCHUNK


def _lse_body(tab_ref, out_ref):
    x = tab_ref[...]
    m = jnp.max(x, axis=1)
    s = jnp.sum(jnp.exp(x - m[:, None]), axis=1)
    out_ref[...] = m + jnp.log(s)


def _loss_body(p_ref, out_ref):
    out_ref[...] = (jnp.sum(p_ref[...]) * (1.0 / N_TOK)).reshape(1, 1)


_NCT = 7
_TAILW = VOCAB - _NCT * 128
_NSPLIT = 4
_SP_TOK = N_TOK // _NSPLIT
_SP_PER_W = _SP_TOK // _NW
_SP_NCHUNK = _SP_PER_W // _CHUNK


def _rowgather_body(idx_hbm, table8_hbm, tail_hbm, out_hbm,
                    idx0_v, idx1_v, fidx0_v, fidx1_v,
                    rows0_v, rows1_v, slab0_v, slab1_v,
                    gsem0, gsem1, ssem0, ssem1, isem0, isem1):
    wid = lax.axis_index("s") * _NC + lax.axis_index("c")
    base = wid * _SP_PER_W

    idx_r = (idx0_v, idx1_v)
    fidx_r = (fidx0_v, fidx1_v)
    rows_r = (rows0_v, rows1_v)
    slab_r = (slab0_v, slab1_v)
    gsem = (gsem0, gsem1)
    ssem = (ssem0, ssem1)
    isem = (isem0, isem1)

    def compute_fidx(b):
        for ct in range(_NCT):
            for o in (0, 16, _CHUNK - 16):
                sl = pl.ds(o, 16)
                fidx_r[b][pl.ds(ct * _CHUNK + o, 16)] = (
                    (idx_r[b][sl] << 3) + ct)

    def start_gathers(b):
        for ct in range(_NCT):
            pltpu.async_copy(
                table8_hbm.at[fidx_r[b].at[pl.ds(ct * _CHUNK, _CHUNK)]],
                rows_r[b].at[:, pl.ds(ct * 128, 128)], gsem[b])
        pltpu.async_copy(tail_hbm.at[idx_r[b]], slab_r[b], gsem[b])

    def wait_gathers(b):
        for ct in range(_NCT):
            pltpu.make_async_copy(
                table8_hbm.at[fidx_r[b].at[pl.ds(ct * _CHUNK, _CHUNK)]],
                rows_r[b].at[:, pl.ds(ct * 128, 128)], gsem[b]).wait()
        pltpu.make_async_copy(tail_hbm.at[idx_r[b]], slab_r[b],
                              gsem[b]).wait()

    def fill_tail(b):
        def row(r, carry):
            for o in (0, 16, 32, 48, 64, _TAILW - 32, _TAILW - 16):
                rows_r[b][r, pl.ds(_NCT * 128 + o, 16)] = (
                    slab_r[b][r, pl.ds(o, 16)])
            return carry
        lax.fori_loop(0, _CHUNK, row, 0)

    def out_write_start(b, tok):
        pltpu.async_copy(rows_r[b], out_hbm.at[pl.ds(tok, _CHUNK)],
                         ssem[b])

    def out_write_wait(b, tok):
        pltpu.make_async_copy(rows_r[b], out_hbm.at[pl.ds(tok, _CHUNK)],
                              ssem[b]).wait()

    for b in (0, 1):
        tok0 = base + b * _CHUNK
        pltpu.sync_copy(idx_hbm.at[pl.ds(tok0, _CHUNK)], idx_r[b])
        compute_fidx(b)
        start_gathers(b)

    def pair(k, carry):
        for b in (0, 1):
            c = 2 * k + b
            tok_c = base + c * _CHUNK
            tok_n = tok_c + 2 * _CHUNK
            wait_gathers(b)
            fill_tail(b)
            out_write_start(b, tok_c)
            pltpu.async_copy(idx_hbm.at[pl.ds(tok_n, _CHUNK)], idx_r[b],
                             isem[b])
            pltpu.make_async_copy(idx_hbm.at[pl.ds(tok_n, _CHUNK)],
                                  idx_r[b], isem[b]).wait()
            compute_fidx(b)
            out_write_wait(b, tok_c)
            start_gathers(b)
        return carry

    lax.fori_loop(0, _SP_NCHUNK // 2 - 1, pair, 0)

    for b in (0, 1):
        c = _SP_NCHUNK - 2 + b
        tok_c = base + c * _CHUNK
        wait_gathers(b)
        fill_tail(b)
        out_write_start(b, tok_c)
        out_write_wait(b, tok_c)


@functools.partial(
    pl.kernel,
    out_type=jax.ShapeDtypeStruct((_SP_TOK, VOCAB), jnp.float32),
    mesh=plsc.VectorSubcoreMesh(core_axis_name="c", subcore_axis_name="s"),
    compiler_params=pltpu.CompilerParams(needs_layout_passes=False),
    scratch_types=(
        [pltpu.VMEM((_CHUNK,), jnp.int32)] * 2
        + [pltpu.VMEM((_NCT * _CHUNK,), jnp.int32)] * 2
        + [pltpu.VMEM((_CHUNK, VOCAB), jnp.float32)] * 2
        + [pltpu.VMEM((_CHUNK, 128), jnp.float32)] * 2
        + [pltpu.SemaphoreType.DMA] * 6
    ),
)
def _sc_rowgather(idx_hbm, table8_hbm, tail_hbm, out_hbm, *scratch):
    _rowgather_body(idx_hbm, table8_hbm, tail_hbm, out_hbm, *scratch)


def _lossgather_body(idx_hbm, tgt_hbm, tabflat_hbm, lse_hbm, part_hbm,
                     idx_v, tgt_v, lseg_v, picked_v, acc_v, sem):
    wid = lax.axis_index("s") * _NC + lax.axis_index("c")
    base = wid * _TOK_PER_W

    pltpu.sync_copy(idx_hbm.at[pl.ds(base, _TOK_PER_W)], idx_v)
    pltpu.sync_copy(tgt_hbm.at[pl.ds(base, _TOK_PER_W)], tgt_v)

    acc_v[...] = jnp.zeros((16,), jnp.float32)

    def chunk(g, carry):
        o = g * _LCHUNK
        sl = pl.ds(o, _LCHUNK)
        pltpu.async_copy(lse_hbm.at[idx_v.at[sl]], lseg_v, sem).wait()
        for j in range(_LCHUNK // 16):
            s16 = pl.ds(o + j * 16, 16)
            idx_v[s16] = idx_v[s16] * VOCAB + tgt_v[s16]
        pltpu.async_copy(tabflat_hbm.at[idx_v.at[sl]], picked_v,
                         sem).wait()
        for j in range(_LCHUNK // 16):
            s16 = pl.ds(j * 16, 16)
            acc_v[...] = acc_v[...] + (lseg_v[s16] - picked_v[s16])
        return carry

    lax.fori_loop(0, _NLCH, chunk, 0)
    pltpu.sync_copy(acc_v, part_hbm.at[wid])


@functools.partial(
    pl.kernel,
    out_type=jax.ShapeDtypeStruct((_NW, 16), jnp.float32),
    mesh=plsc.VectorSubcoreMesh(core_axis_name="c", subcore_axis_name="s"),
    compiler_params=pltpu.CompilerParams(use_tc_tiling_on_sc=False,
                                         needs_layout_passes=False),
    scratch_types=(
        [pltpu.VMEM((_TOK_PER_W,), jnp.int32)] * 2
        + [pltpu.VMEM((_LCHUNK,), jnp.float32)] * 2
        + [pltpu.VMEM((16,), jnp.float32)]
        + [pltpu.SemaphoreType.DMA]
    ),
)
def _sc_lossgather(idx_hbm, tgt_hbm, tabflat_hbm, lse_hbm, part_hbm,
                   *scratch):
    _lossgather_body(idx_hbm, tgt_hbm, tabflat_hbm, lse_hbm, part_hbm,
                     *scratch)


def kernel(idx, targets, table):
    idx_f = idx.reshape(-1).astype(jnp.int32)
    tgt_f = targets.reshape(-1).astype(jnp.int32)
    table = table.astype(jnp.float32)

    tab_pad = jnp.pad(table, ((0, 0), (0, CPAD - VOCAB)),
                      constant_values=-1e30)

    lse_table = pl.pallas_call(
        _lse_body,
        out_shape=jax.ShapeDtypeStruct((VOCAB,), jnp.float32),
    )(tab_pad)

    tab8 = tab_pad.reshape(VOCAB * 8, 128)
    tab_tail = jnp.pad(table[:, _NCT * 128:], ((0, 0), (0, 128 - _TAILW)))
    parts = [
        _sc_rowgather(lax.dynamic_slice(idx_f, (s * _SP_TOK,), (_SP_TOK,)),
                      tab8, tab_tail)
        for s in range(_NSPLIT)
    ]
    logits2 = jnp.concatenate(parts, axis=0)

    tab_flat = jnp.pad(table.reshape(-1), (0, 8))
    partials = _sc_lossgather(idx_f, tgt_f, tab_flat, lse_table)

    loss = pl.pallas_call(
        _loss_body,
        out_shape=jax.ShapeDtypeStruct((1, 1), jnp.float32),
    )(partials)[0, 0]

    return (logits2, loss)


# speedup vs baseline: 1.0195x; 1.0195x over previous
import functools

import jax
import jax.numpy as jnp
from jax import lax
from jax.experimental import pallas as pl
from jax.experimental.pallas import tpu as pltpu
from jax.experimental.pallas import tpu_sc as plsc

VOCAB = 1000
CPAD = 1024
B, T = 1024, 200
N_TOK = B * T

_NC, _NS = 2, 16
_NW = _NC * _NS
_TOK_PER_W = N_TOK // _NW

_CHUNK = 40
_NCHUNK = _TOK_PER_W // _CHUNK

_LCHUNK = 128
_NLCH = _TOK_PER_W // _LCHUNK


def _lse_body(tab_ref, out_ref):
    x = tab_ref[...]
    m = jnp.max(x, axis=1)
    s = jnp.sum(jnp.exp(x - m[:, None]), axis=1)
    out_ref[...] = m + jnp.log(s)


def _loss_body(p_ref, out_ref):
    out_ref[...] = (jnp.sum(p_ref[...]) * (1.0 / N_TOK)).reshape(1, 1)


_NCT = 7
_TAILW = VOCAB - _NCT * 128
_NSPLIT = 4
_SP_TOK = N_TOK // _NSPLIT
_SP_PER_W = _SP_TOK // _NW
_SP_NCHUNK = _SP_PER_W // _CHUNK


def _rowgather_body(idx_hbm, table8_hbm, tail_hbm, out_hbm,
                    idx0_v, idx1_v, fidx0_v, fidx1_v,
                    rows0_v, rows1_v, slab0_v, slab1_v,
                    gsem0, gsem1, ssem0, ssem1, isem0, isem1):
    wid = lax.axis_index("s") * _NC + lax.axis_index("c")
    base = wid * _SP_PER_W

    idx_r = (idx0_v, idx1_v)
    fidx_r = (fidx0_v, fidx1_v)
    rows_r = (rows0_v, rows1_v)
    slab_r = (slab0_v, slab1_v)
    gsem = (gsem0, gsem1)
    ssem = (ssem0, ssem1)
    isem = (isem0, isem1)

    def compute_fidx(b):
        for ct in range(_NCT):
            for o in (0, 16, _CHUNK - 16):
                sl = pl.ds(o, 16)
                fidx_r[b][pl.ds(ct * _CHUNK + o, 16)] = (
                    (idx_r[b][sl] << 3) + ct)

    def start_gathers(b):
        for ct in range(_NCT):
            pltpu.async_copy(
                table8_hbm.at[fidx_r[b].at[pl.ds(ct * _CHUNK, _CHUNK)]],
                rows_r[b].at[:, pl.ds(ct * 128, 128)], gsem[b])
        pltpu.async_copy(tail_hbm.at[idx_r[b]], slab_r[b], gsem[b])

    def wait_gathers(b):
        for ct in range(_NCT):
            pltpu.make_async_copy(
                table8_hbm.at[fidx_r[b].at[pl.ds(ct * _CHUNK, _CHUNK)]],
                rows_r[b].at[:, pl.ds(ct * 128, 128)], gsem[b]).wait()
        pltpu.make_async_copy(tail_hbm.at[idx_r[b]], slab_r[b],
                              gsem[b]).wait()

    def fill_tail(b):
        def row(r, carry):
            for o in (0, 16, 32, 48, 64, _TAILW - 32, _TAILW - 16):
                rows_r[b][r, pl.ds(_NCT * 128 + o, 16)] = (
                    slab_r[b][r, pl.ds(o, 16)])
            return carry
        lax.fori_loop(0, _CHUNK, row, 0)

    def out_write_start(b, tok):
        pltpu.async_copy(rows_r[b], out_hbm.at[pl.ds(tok, _CHUNK)],
                         ssem[b])

    def out_write_wait(b, tok):
        pltpu.make_async_copy(rows_r[b], out_hbm.at[pl.ds(tok, _CHUNK)],
                              ssem[b]).wait()

    for b in (0, 1):
        tok0 = base + b * _CHUNK
        pltpu.sync_copy(idx_hbm.at[pl.ds(tok0, _CHUNK)], idx_r[b])
        compute_fidx(b)
        start_gathers(b)

    def pair(k, carry):
        for b in (0, 1):
            c = 2 * k + b
            tok_c = base + c * _CHUNK
            tok_n = tok_c + 2 * _CHUNK
            wait_gathers(b)
            fill_tail(b)
            out_write_start(b, tok_c)
            pltpu.async_copy(idx_hbm.at[pl.ds(tok_n, _CHUNK)], idx_r[b],
                             isem[b])
            pltpu.make_async_copy(idx_hbm.at[pl.ds(tok_n, _CHUNK)],
                                  idx_r[b], isem[b]).wait()
            compute_fidx(b)
            out_write_wait(b, tok_c)
            start_gathers(b)
        return carry

    lax.fori_loop(0, _SP_NCHUNK // 2 - 1, pair, 0)

    for b in (0, 1):
        c = _SP_NCHUNK - 2 + b
        tok_c = base + c * _CHUNK
        wait_gathers(b)
        fill_tail(b)
        out_write_start(b, tok_c)
        out_write_wait(b, tok_c)


@functools.partial(
    pl.kernel,
    out_type=jax.ShapeDtypeStruct((_SP_TOK, VOCAB), jnp.float32),
    mesh=plsc.VectorSubcoreMesh(core_axis_name="c", subcore_axis_name="s"),
    compiler_params=pltpu.CompilerParams(needs_layout_passes=False),
    scratch_types=(
        [pltpu.VMEM((_CHUNK,), jnp.int32)] * 2
        + [pltpu.VMEM((_NCT * _CHUNK,), jnp.int32)] * 2
        + [pltpu.VMEM((_CHUNK, VOCAB), jnp.float32)] * 2
        + [pltpu.VMEM((_CHUNK, 128), jnp.float32)] * 2
        + [pltpu.SemaphoreType.DMA] * 6
    ),
)
def _sc_rowgather(idx_hbm, table8_hbm, tail_hbm, out_hbm, *scratch):
    _rowgather_body(idx_hbm, table8_hbm, tail_hbm, out_hbm, *scratch)


def _lossgather_body(idx_hbm, tgt_hbm, tabflat_hbm, lse_hbm, part_hbm,
                     idx_v, tgt_v, lseg_v, picked_v, acc_v, sem):
    wid = lax.axis_index("s") * _NC + lax.axis_index("c")
    base = wid * _TOK_PER_W

    pltpu.sync_copy(idx_hbm.at[pl.ds(base, _TOK_PER_W)], idx_v)
    pltpu.sync_copy(tgt_hbm.at[pl.ds(base, _TOK_PER_W)], tgt_v)

    acc_v[...] = jnp.zeros((16,), jnp.float32)

    def chunk(g, carry):
        o = g * _LCHUNK
        sl = pl.ds(o, _LCHUNK)
        pltpu.async_copy(lse_hbm.at[idx_v.at[sl]], lseg_v, sem).wait()
        for j in range(_LCHUNK // 16):
            s16 = pl.ds(o + j * 16, 16)
            idx_v[s16] = idx_v[s16] * VOCAB + tgt_v[s16]
        pltpu.async_copy(tabflat_hbm.at[idx_v.at[sl]], picked_v,
                         sem).wait()
        for j in range(_LCHUNK // 16):
            s16 = pl.ds(j * 16, 16)
            acc_v[...] = acc_v[...] + (lseg_v[s16] - picked_v[s16])
        return carry

    lax.fori_loop(0, _NLCH, chunk, 0)
    pltpu.sync_copy(acc_v, part_hbm.at[wid])


@functools.partial(
    pl.kernel,
    out_type=jax.ShapeDtypeStruct((_NW, 16), jnp.float32),
    mesh=plsc.VectorSubcoreMesh(core_axis_name="c", subcore_axis_name="s"),
    compiler_params=pltpu.CompilerParams(use_tc_tiling_on_sc=False,
                                         needs_layout_passes=False),
    scratch_types=(
        [pltpu.VMEM((_TOK_PER_W,), jnp.int32)] * 2
        + [pltpu.VMEM((_LCHUNK,), jnp.float32)] * 2
        + [pltpu.VMEM((16,), jnp.float32)]
        + [pltpu.SemaphoreType.DMA]
    ),
)
def _sc_lossgather(idx_hbm, tgt_hbm, tabflat_hbm, lse_hbm, part_hbm,
                   *scratch):
    _lossgather_body(idx_hbm, tgt_hbm, tabflat_hbm, lse_hbm, part_hbm,
                     *scratch)


def kernel(idx, targets, table):
    idx_f = idx.reshape(-1).astype(jnp.int32)
    tgt_f = targets.reshape(-1).astype(jnp.int32)
    table = table.astype(jnp.float32)

    tab_pad = jnp.pad(table, ((0, 0), (0, CPAD - VOCAB)),
                      constant_values=-1e30)

    lse_table = pl.pallas_call(
        _lse_body,
        out_shape=jax.ShapeDtypeStruct((VOCAB,), jnp.float32),
    )(tab_pad)

    tab8 = tab_pad.reshape(VOCAB * 8, 128)
    tab_tail = jnp.pad(table[:, _NCT * 128:], ((0, 0), (0, 128 - _TAILW)))
    logits2 = jnp.empty((N_TOK, VOCAB), jnp.float32)
    for s in range(_NSPLIT):
        part = _sc_rowgather(
            lax.dynamic_slice(idx_f, (s * _SP_TOK,), (_SP_TOK,)),
            tab8, tab_tail)
        logits2 = lax.dynamic_update_slice(logits2, part, (s * _SP_TOK, 0))

    tab_flat = jnp.pad(table.reshape(-1), (0, 8))
    partials = _sc_lossgather(idx_f, tgt_f, tab_flat, lse_table)

    loss = pl.pallas_call(
        _loss_body,
        out_shape=jax.ShapeDtypeStruct((1, 1), jnp.float32),
    )(partials)[0, 0]

    return (logits2, loss)


# speedup vs baseline: 1.4326x; 1.4052x over previous
import functools

import jax
import jax.numpy as jnp
from jax import lax
from jax.experimental import pallas as pl
from jax.experimental.pallas import tpu as pltpu
from jax.experimental.pallas import tpu_sc as plsc

VOCAB = 1000
CPAD = 1024
B, T = 1024, 200
N_TOK = B * T

_NC, _NS = 2, 16
_NW = _NC * _NS
_TOK_PER_W = N_TOK // _NW

_CHUNK = 40
_NCHUNK = _TOK_PER_W // _CHUNK

_LCHUNK = 128
_NLCH = _TOK_PER_W // _LCHUNK


def _lse_body(tab_ref, out_ref):
    x = tab_ref[...]
    m = jnp.max(x, axis=1)
    s = jnp.sum(jnp.exp(x - m[:, None]), axis=1)
    out_ref[...] = m + jnp.log(s)


def _loss_body(p_ref, out_ref):
    out_ref[...] = (jnp.sum(p_ref[...]) * (1.0 / N_TOK)).reshape(1, 1)


_NCT = 7
_TAILW = VOCAB - _NCT * 128
_SP_TOK = N_TOK
_SP_PER_W = _SP_TOK // _NW
_SP_NCHUNK = _SP_PER_W // _CHUNK


def _rowgather_body(idx_hbm, tgt_hbm, table8_hbm, tail_hbm, tabflat_hbm,
                    lse_hbm, out_hbm, part_hbm,
                    idx0_v, idx1_v, tgt0_v, tgt1_v, fidx0_v, fidx1_v,
                    fi0_v, fi1_v, rows0_v, rows1_v, slab0_v, slab1_v,
                    lsg0_v, lsg1_v, pck0_v, pck1_v, acc_v,
                    gsem0, gsem1, ssem0, ssem1, isem0, isem1):
    wid = lax.axis_index("s") * _NC + lax.axis_index("c")
    base = wid * _SP_PER_W

    idx_r = (idx0_v, idx1_v)
    tgt_r = (tgt0_v, tgt1_v)
    fidx_r = (fidx0_v, fidx1_v)
    fi_r = (fi0_v, fi1_v)
    rows_r = (rows0_v, rows1_v)
    slab_r = (slab0_v, slab1_v)
    lsg_r = (lsg0_v, lsg1_v)
    pck_r = (pck0_v, pck1_v)
    gsem = (gsem0, gsem1)
    ssem = (ssem0, ssem1)
    isem = (isem0, isem1)

    _SLICES = (0, 16, _CHUNK - 16)

    def compute_fidx(b):
        for ct in range(_NCT):
            for o in _SLICES:
                sl = pl.ds(o, 16)
                fidx_r[b][pl.ds(ct * _CHUNK + o, 16)] = (
                    (idx_r[b][sl] << 3) + ct)
        for o in _SLICES:
            sl = pl.ds(o, 16)
            fi_r[b][sl] = idx_r[b][sl] * VOCAB + tgt_r[b][sl]

    def start_gathers(b):
        for ct in range(_NCT):
            pltpu.async_copy(
                table8_hbm.at[fidx_r[b].at[pl.ds(ct * _CHUNK, _CHUNK)]],
                rows_r[b].at[:, pl.ds(ct * 128, 128)], gsem[b])
        pltpu.async_copy(tail_hbm.at[idx_r[b]], slab_r[b], gsem[b])
        pltpu.async_copy(lse_hbm.at[idx_r[b]], lsg_r[b], gsem[b])
        pltpu.async_copy(tabflat_hbm.at[fi_r[b]], pck_r[b], gsem[b])

    def wait_gathers(b):
        for ct in range(_NCT):
            pltpu.make_async_copy(
                table8_hbm.at[fidx_r[b].at[pl.ds(ct * _CHUNK, _CHUNK)]],
                rows_r[b].at[:, pl.ds(ct * 128, 128)], gsem[b]).wait()
        pltpu.make_async_copy(tail_hbm.at[idx_r[b]], slab_r[b],
                              gsem[b]).wait()
        pltpu.make_async_copy(lse_hbm.at[idx_r[b]], lsg_r[b],
                              gsem[b]).wait()
        pltpu.make_async_copy(tabflat_hbm.at[fi_r[b]], pck_r[b],
                              gsem[b]).wait()

    def fill_tail(b):
        def row(r, carry):
            for o in (0, 16, 32, 48, 64, _TAILW - 32, _TAILW - 16):
                rows_r[b][r, pl.ds(_NCT * 128 + o, 16)] = (
                    slab_r[b][r, pl.ds(o, 16)])
            return carry
        lax.fori_loop(0, _CHUNK, row, 0)

    def acc_update(b):
        lanes = lax.iota(jnp.int32, 16)
        for o in _SLICES:
            sl = pl.ds(o, 16)
            d = lsg_r[b][sl] - pck_r[b][sl]
            if o == _CHUNK - 16:
                d = jnp.where(lanes >= 8, d, 0.0)
            acc_v[...] = acc_v[...] + d

    def out_write_start(b, tok):
        pltpu.async_copy(rows_r[b], out_hbm.at[pl.ds(tok, _CHUNK)],
                         ssem[b])

    def out_write_wait(b, tok):
        pltpu.make_async_copy(rows_r[b], out_hbm.at[pl.ds(tok, _CHUNK)],
                              ssem[b]).wait()

    acc_v[...] = jnp.zeros((16,), jnp.float32)

    for b in (0, 1):
        tok0 = base + b * _CHUNK
        pltpu.sync_copy(idx_hbm.at[pl.ds(tok0, _CHUNK)], idx_r[b])
        pltpu.sync_copy(tgt_hbm.at[pl.ds(tok0, _CHUNK)], tgt_r[b])
        compute_fidx(b)
        start_gathers(b)

    def pair(k, carry):
        for b in (0, 1):
            c = 2 * k + b
            tok_c = base + c * _CHUNK
            tok_n = tok_c + 2 * _CHUNK
            wait_gathers(b)
            fill_tail(b)
            out_write_start(b, tok_c)
            acc_update(b)
            pltpu.async_copy(idx_hbm.at[pl.ds(tok_n, _CHUNK)], idx_r[b],
                             isem[b])
            pltpu.async_copy(tgt_hbm.at[pl.ds(tok_n, _CHUNK)], tgt_r[b],
                             isem[b])
            pltpu.make_async_copy(idx_hbm.at[pl.ds(tok_n, _CHUNK)],
                                  idx_r[b], isem[b]).wait()
            pltpu.make_async_copy(tgt_hbm.at[pl.ds(tok_n, _CHUNK)],
                                  tgt_r[b], isem[b]).wait()
            compute_fidx(b)
            out_write_wait(b, tok_c)
            start_gathers(b)
        return carry

    lax.fori_loop(0, _SP_NCHUNK // 2 - 1, pair, 0)

    for b in (0, 1):
        c = _SP_NCHUNK - 2 + b
        tok_c = base + c * _CHUNK
        wait_gathers(b)
        fill_tail(b)
        out_write_start(b, tok_c)
        acc_update(b)
        out_write_wait(b, tok_c)

    pltpu.sync_copy(acc_v, part_hbm.at[pl.ds(wid * 16, 16)])


@functools.partial(
    pl.kernel,
    out_type=[
        jax.ShapeDtypeStruct((_SP_TOK, VOCAB), jnp.float32),
        jax.ShapeDtypeStruct((_NW * 16,), jnp.float32),
    ],
    mesh=plsc.VectorSubcoreMesh(core_axis_name="c", subcore_axis_name="s"),
    compiler_params=pltpu.CompilerParams(needs_layout_passes=False),
    scratch_types=(
        [pltpu.VMEM((_CHUNK,), jnp.int32)] * 4
        + [pltpu.VMEM((_NCT * _CHUNK,), jnp.int32)] * 2
        + [pltpu.VMEM((_CHUNK,), jnp.int32)] * 2
        + [pltpu.VMEM((_CHUNK, VOCAB), jnp.float32)] * 2
        + [pltpu.VMEM((_CHUNK, 128), jnp.float32)] * 2
        + [pltpu.VMEM((_CHUNK,), jnp.float32)] * 4
        + [pltpu.VMEM((16,), jnp.float32)]
        + [pltpu.SemaphoreType.DMA] * 6
    ),
)
def _sc_rowgather(idx_hbm, tgt_hbm, table8_hbm, tail_hbm, tabflat_hbm,
                  lse_hbm, out_hbm, part_hbm, *scratch):
    _rowgather_body(idx_hbm, tgt_hbm, table8_hbm, tail_hbm, tabflat_hbm,
                    lse_hbm, out_hbm, part_hbm, *scratch)


def kernel(idx, targets, table):
    idx_f = idx.reshape(-1).astype(jnp.int32)
    tgt_f = targets.reshape(-1).astype(jnp.int32)
    table = table.astype(jnp.float32)

    tab_pad = jnp.pad(table, ((0, 0), (0, CPAD - VOCAB)),
                      constant_values=-1e30)

    lse_table = pl.pallas_call(
        _lse_body,
        out_shape=jax.ShapeDtypeStruct((VOCAB,), jnp.float32),
    )(tab_pad)

    tab8 = tab_pad.reshape(VOCAB * 8, 128)
    tab_tail = jnp.pad(table[:, _NCT * 128:], ((0, 0), (0, 128 - _TAILW)))
    tab_flat = jnp.pad(table.reshape(-1), (0, 8))
    logits2, partials = _sc_rowgather(idx_f, tgt_f, tab8, tab_tail,
                                      tab_flat, lse_table)

    loss = pl.pallas_call(
        _loss_body,
        out_shape=jax.ShapeDtypeStruct((1, 1), jnp.float32),
    )(partials)[0, 0]

    return (logits2, loss)
